# baseline (speedup 1.0000x reference)
import functools

import jax
import jax.numpy as jnp
from jax import lax
from jax.experimental import pallas as pl
from jax.experimental.pallas import tpu as pltpu
from jax.experimental.pallas import tpu_sc as plsc

CHUNK = 128
HALF = CHUNK // 2


@functools.lru_cache(maxsize=None)
def _build_gather(B, DT, V):
    info = plsc.get_sparse_core_info()
    NC, NS = info.num_cores, info.num_subcores
    NW = NC * NS
    per_w = B // NW
    assert per_w * NW == B and per_w % CHUNK == 0
    k = per_w // CHUNK
    assert k % 2 == 0
    m = k // 2 - 1

    mesh = plsc.VectorSubcoreMesh(core_axis_name="c", subcore_axis_name="s")

    @functools.partial(
        pl.kernel,
        mesh=mesh,
        out_type=jax.ShapeDtypeStruct((B // 2, 2 * DT), jnp.float32),
        scratch_types=[
            pltpu.VMEM((k, 2, HALF), jnp.int32),
            pltpu.VMEM((2, HALF, DT), jnp.float32),
            pltpu.VMEM((2, HALF, DT), jnp.float32),
            pltpu.SemaphoreType.DMA,
            pltpu.SemaphoreType.DMA,
            pltpu.SemaphoreType.DMA,
            pltpu.SemaphoreType.DMA,
        ],
        compiler_params=pltpu.CompilerParams(use_tc_tiling_on_sc=False),
    )
    def sc_kernel(idx_hbm, table_hbm, g_hbm, idx_v, buf0, buf1, si0, si1, so0, so1):
        wid = lax.axis_index("s") * NC + lax.axis_index("c")
        base = wid * per_w // 2
        pltpu.sync_copy(idx_hbm.at[wid], idx_v)

        def start_in(j, buf, sem):
            pltpu.async_copy(table_hbm.at[idx_v.at[j, 0]], buf.at[0], sem)
            pltpu.async_copy(table_hbm.at[idx_v.at[j, 1]], buf.at[1], sem)

        def start_out(j, buf, sem):
            rows = pl.ds(base + j * HALF, HALF)
            pltpu.async_copy(buf.at[0], g_hbm.at[rows, pl.ds(0, DT)], sem)
            pltpu.async_copy(buf.at[1], g_hbm.at[rows, pl.ds(DT, DT)], sem)

        def wait_in(sem, buf):
            for h in (0, 1):
                pltpu.make_async_copy(
                    table_hbm.at[idx_v.at[0, 0]], buf.at[h], sem
                ).wait()

        def wait_out(sem, buf):
            for h in (0, 1):
                pltpu.make_async_copy(
                    buf.at[h], g_hbm.at[pl.ds(base, HALF), pl.ds(0, DT)], sem
                ).wait()

        start_in(0, buf0, si0)
        start_in(1, buf1, si1)

        def body(i, _):
            j = 2 * i
            wait_in(si0, buf0)
            start_out(j, buf0, so0)
            wait_in(si1, buf1)
            start_out(j + 1, buf1, so1)
            wait_out(so0, buf0)
            start_in(j + 2, buf0, si0)
            wait_out(so1, buf1)
            start_in(j + 3, buf1, si1)
            return ()

        lax.fori_loop(0, m, body, (), unroll=False)

        wait_in(si0, buf0)
        start_out(k - 2, buf0, so0)
        wait_in(si1, buf1)
        start_out(k - 1, buf1, so1)
        wait_out(so0, buf0)
        wait_out(so1, buf1)

    return sc_kernel, NW, k


@functools.lru_cache(maxsize=None)
def _build_concat(S0, S1, DE, DT, BS, S1B):
    DO = DE + DT

    def concat_body(elmo_ref, g_ref, out_ref):
        e = elmo_ref[...]
        et = jnp.transpose(e, (0, 2, 1))
        g = g_ref[...]
        g4 = g.reshape(S1B, BS // 2, 2, DT)
        gt = jnp.transpose(g4, (0, 3, 1, 2)).reshape(S1B, DT, BS)
        out_ref[...] = jnp.concatenate((et, gt), axis=1)

    return pl.pallas_call(
        concat_body,
        grid=(S0 // BS, S1 // S1B),
        in_specs=[
            pl.BlockSpec((S1B, BS, DE), lambda i, j: (j, i, 0)),
            pl.BlockSpec((S1B, BS // 2, 2 * DT), lambda i, j: (j, i, 0)),
        ],
        out_specs=pl.BlockSpec((S1B, DO, BS), lambda i, j: (j, 0, i)),
        out_shape=jax.ShapeDtypeStruct((S1, DO, S0), jnp.float32),
    )


def kernel(elmo_emb, inp, table):
    S0, S1, DE = elmo_emb.shape
    V, DT = table.shape
    B = S0 * S1
    sc_gather, NW, k = _build_gather(B, DT, V)
    idx = (
        inp.transpose(1, 0)
        .reshape(NW, k, HALF, 2)
        .transpose(0, 1, 3, 2)
        .astype(jnp.int32)
    )
    g = sc_gather(idx, table)
    g3 = g.reshape(S1, S0 // 2, 2 * DT)
    concat = _build_concat(S0, S1, DE, DT, 128, 10)
    elmo_t = elmo_emb.transpose(1, 0, 2)
    out_t = concat(elmo_t, g3)
    return out_t.transpose(2, 0, 1)


# speedup vs baseline: 2.9256x; 2.9256x over previous
import functools

import jax
import jax.numpy as jnp
from jax import lax
from jax.experimental import pallas as pl
from jax.experimental.pallas import tpu as pltpu
from jax.experimental.pallas import tpu_sc as plsc

CHUNK = 128
HALF = CHUNK // 2


@functools.lru_cache(maxsize=None)
def _build_gather(B, DT, V):
    info = plsc.get_sparse_core_info()
    NC, NS = info.num_cores, info.num_subcores
    NW = NC * NS
    per_w = B // NW
    assert per_w * NW == B and per_w % CHUNK == 0
    k = per_w // CHUNK
    assert k % 2 == 0
    m = k // 2 - 1

    mesh = plsc.VectorSubcoreMesh(core_axis_name="c", subcore_axis_name="s")

    @functools.partial(
        pl.kernel,
        mesh=mesh,
        out_type=jax.ShapeDtypeStruct((B // 2, 2 * DT), jnp.float32),
        scratch_types=[
            pltpu.VMEM((k, 2, HALF), jnp.int32),
            pltpu.VMEM((2, HALF, DT), jnp.float32),
            pltpu.VMEM((2, HALF, DT), jnp.float32),
            pltpu.SemaphoreType.DMA,
            pltpu.SemaphoreType.DMA,
            pltpu.SemaphoreType.DMA,
            pltpu.SemaphoreType.DMA,
        ],
        compiler_params=pltpu.CompilerParams(use_tc_tiling_on_sc=False),
    )
    def sc_kernel(idx_hbm, table_hbm, g_hbm, idx_v, buf0, buf1, si0, si1, so0, so1):
        wid = lax.axis_index("s") * NC + lax.axis_index("c")
        base = wid * per_w // 2
        pltpu.sync_copy(idx_hbm.at[wid], idx_v)

        def start_in(j, buf, sem):
            pltpu.async_copy(table_hbm.at[idx_v.at[j, 0]], buf.at[0], sem)
            pltpu.async_copy(table_hbm.at[idx_v.at[j, 1]], buf.at[1], sem)

        def start_out(j, buf, sem):
            rows = pl.ds(base + j * HALF, HALF)
            pltpu.async_copy(buf.at[0], g_hbm.at[rows, pl.ds(0, DT)], sem)
            pltpu.async_copy(buf.at[1], g_hbm.at[rows, pl.ds(DT, DT)], sem)

        def wait_in(sem, buf):
            for h in (0, 1):
                pltpu.make_async_copy(
                    table_hbm.at[idx_v.at[0, 0]], buf.at[h], sem
                ).wait()

        def wait_out(sem, buf):
            for h in (0, 1):
                pltpu.make_async_copy(
                    buf.at[h], g_hbm.at[pl.ds(base, HALF), pl.ds(0, DT)], sem
                ).wait()

        start_in(0, buf0, si0)
        start_in(1, buf1, si1)

        def body(i, _):
            j = 2 * i
            wait_in(si0, buf0)
            start_out(j, buf0, so0)
            wait_in(si1, buf1)
            start_out(j + 1, buf1, so1)
            wait_out(so0, buf0)
            start_in(j + 2, buf0, si0)
            wait_out(so1, buf1)
            start_in(j + 3, buf1, si1)
            return ()

        lax.fori_loop(0, m, body, (), unroll=False)

        wait_in(si0, buf0)
        start_out(k - 2, buf0, so0)
        wait_in(si1, buf1)
        start_out(k - 1, buf1, so1)
        wait_out(so0, buf0)
        wait_out(so1, buf1)

    return sc_kernel, NW, k


@functools.lru_cache(maxsize=None)
def _build_concat(S0, S1, DE, DT, BS, S1B):
    DO = DE + DT

    def concat_body(elmo_ref, g_ref, out_ref):
        e = elmo_ref[...]
        et = jnp.transpose(e, (0, 2, 1))
        g = g_ref[...]
        gt = jnp.transpose(g, (0, 2, 1))
        gt = gt.reshape(S1B, DT, BS)
        out_ref[...] = jnp.concatenate((et, gt), axis=1)

    return pl.pallas_call(
        concat_body,
        grid=(S0 // BS, S1 // S1B),
        in_specs=[
            pl.BlockSpec((S1B, BS, DE), lambda i, j: (j, i, 0)),
            pl.BlockSpec((S1B // 2, BS, 2 * DT), lambda i, j: (j, i, 0)),
        ],
        out_specs=pl.BlockSpec((S1B, DO, BS), lambda i, j: (j, 0, i)),
        out_shape=jax.ShapeDtypeStruct((S1, DO, S0), jnp.float32),
    )


def kernel(elmo_emb, inp, table):
    S0, S1, DE = elmo_emb.shape
    V, DT = table.shape
    B = S0 * S1
    sc_gather, NW, k = _build_gather(B, DT, V)
    ip = inp.transpose(1, 0).reshape(S1 // 2, 2, S0).astype(jnp.int32)
    ev = ip[:, 0, :].reshape(NW, k, HALF)
    od = ip[:, 1, :].reshape(NW, k, HALF)
    idx = jnp.stack((ev, od), axis=2)
    g = sc_gather(idx, table)
    g3 = g.reshape(S1 // 2, S0, 2 * DT)
    concat = _build_concat(S0, S1, DE, DT, 128, 10)
    elmo_t = elmo_emb.transpose(1, 0, 2)
    out_t = concat(elmo_t, g3)
    return out_t.transpose(2, 0, 1)


# speedup vs baseline: 3.1234x; 1.0676x over previous
import functools

import jax
import jax.numpy as jnp
from jax import lax
from jax.experimental import pallas as pl
from jax.experimental.pallas import tpu as pltpu
from jax.experimental.pallas import tpu_sc as plsc

CHUNK = 128
HALF = CHUNK // 2


@functools.lru_cache(maxsize=None)
def _build_gather(B, DT, V2):
    info = plsc.get_sparse_core_info()
    NC, NS = info.num_cores, info.num_subcores
    NW = NC * NS
    per_w = B // NW
    assert per_w * NW == B and per_w % CHUNK == 0
    k = per_w // CHUNK
    assert k % 2 == 0
    m = k // 2 - 1

    mesh = plsc.VectorSubcoreMesh(core_axis_name="c", subcore_axis_name="s")

    @functools.partial(
        pl.kernel,
        mesh=mesh,
        out_type=jax.ShapeDtypeStruct((B // 2, 2 * DT), jnp.float32),
        scratch_types=[
            pltpu.VMEM((k, 2, HALF), jnp.int32),
            pltpu.VMEM((2, HALF, DT), jnp.float32),
            pltpu.VMEM((2, HALF, DT), jnp.float32),
            pltpu.SemaphoreType.DMA,
            pltpu.SemaphoreType.DMA,
            pltpu.SemaphoreType.DMA,
            pltpu.SemaphoreType.DMA,
        ],
        compiler_params=pltpu.CompilerParams(use_tc_tiling_on_sc=False),
    )
    def sc_kernel(idx_hbm, table_hbm, g_hbm, idx_v, buf0, buf1, si0, si1, so0, so1):
        wid = lax.axis_index("s") * NC + lax.axis_index("c")
        base = wid * per_w // 2
        pltpu.sync_copy(idx_hbm.at[wid], idx_v)

        def start_in(j, buf, sem):
            pltpu.async_copy(table_hbm.at[idx_v.at[j, 0]], buf.at[0], sem)
            pltpu.async_copy(table_hbm.at[idx_v.at[j, 1]], buf.at[1], sem)

        def start_out(j, buf, sem):
            rows = pl.ds(base + j * HALF, HALF)
            pltpu.async_copy(buf.at[0], g_hbm.at[rows, pl.ds(0, DT)], sem)
            pltpu.async_copy(buf.at[1], g_hbm.at[rows, pl.ds(DT, DT)], sem)

        def wait_in(sem, buf):
            for h in (0, 1):
                pltpu.make_async_copy(
                    table_hbm.at[idx_v.at[0, 0]], buf.at[h], sem
                ).wait()

        def wait_out(sem, buf):
            for h in (0, 1):
                pltpu.make_async_copy(
                    buf.at[h], g_hbm.at[pl.ds(base, HALF), pl.ds(0, DT)], sem
                ).wait()

        start_in(0, buf0, si0)
        start_in(1, buf1, si1)

        def body(i, _):
            j = 2 * i
            wait_in(si0, buf0)
            start_out(j, buf0, so0)
            wait_in(si1, buf1)
            start_out(j + 1, buf1, so1)
            wait_out(so0, buf0)
            start_in(j + 2, buf0, si0)
            wait_out(so1, buf1)
            start_in(j + 3, buf1, si1)
            return ()

        lax.fori_loop(0, m, body, (), unroll=False)

        wait_in(si0, buf0)
        start_out(k - 2, buf0, so0)
        wait_in(si1, buf1)
        start_out(k - 1, buf1, so1)
        wait_out(so0, buf0)
        wait_out(so1, buf1)

    return sc_kernel, NW, k


@functools.lru_cache(maxsize=None)
def _build_concat(S0, S1, DE, DT, BS, S1B):
    DO = DE + DT

    def concat_body(elmo_ref, g_ref, out_ref):
        e = elmo_ref[...]
        et = jnp.transpose(e, (0, 2, 1))
        g = g_ref[...]
        gt = jnp.transpose(g, (0, 2, 1))
        gt = gt.reshape(S1B, DT, BS)
        out_ref[...] = jnp.concatenate((et, gt), axis=1)

    return pl.pallas_call(
        concat_body,
        grid=(S0 // BS, S1 // S1B),
        in_specs=[
            pl.BlockSpec((S1B, BS, DE), lambda i, j: (j, i, 0)),
            pl.BlockSpec((S1B // 2, BS, 2 * DT), lambda i, j: (j, i, 0)),
        ],
        out_specs=pl.BlockSpec((S1B, DO, BS), lambda i, j: (j, 0, i)),
        out_shape=jax.ShapeDtypeStruct((S1, DO, S0), jnp.float32),
    )


@functools.lru_cache(maxsize=None)
def _build_detile(V, DT, BC):
    import math

    R = BC // 2
    grid = math.ceil(V / BC)

    def detile_body(t_ref, out_ref):
        a = t_ref[...]
        at = jnp.transpose(a, (1, 0))
        at3 = at.reshape(R, 2, DT)
        out_ref[...] = jnp.concatenate((at3[:, 0, :], at3[:, 1, :]), axis=1)

    return pl.pallas_call(
        detile_body,
        grid=(grid,),
        in_specs=[pl.BlockSpec((DT, BC), lambda i: (0, i))],
        out_specs=pl.BlockSpec((R, 2 * DT), lambda i: (i, 0)),
        out_shape=jax.ShapeDtypeStruct((V // 2, 2 * DT), jnp.float32),
    )


def kernel(elmo_emb, inp, table):
    S0, S1, DE = elmo_emb.shape
    V, DT = table.shape
    B = S0 * S1
    sc_gather, NW, k = _build_gather(B, DT, V)
    detile = _build_detile(V, DT, 2048)
    table_lin = detile(table.transpose(1, 0)).reshape(V, DT)
    ip = inp.transpose(1, 0).reshape(S1 // 2, 2, S0).astype(jnp.int32)
    ev = ip[:, 0, :].reshape(NW, k, HALF)
    od = ip[:, 1, :].reshape(NW, k, HALF)
    idx = jnp.stack((ev, od), axis=2)
    g = sc_gather(idx, table_lin)
    g3 = g.reshape(S1 // 2, S0, 2 * DT)
    concat = _build_concat(S0, S1, DE, DT, 128, 10)
    elmo_t = elmo_emb.transpose(1, 0, 2)
    out_t = concat(elmo_t, g3)
    return out_t.transpose(2, 0, 1)


# speedup vs baseline: 3.4925x; 1.1182x over previous
import functools

import jax
import jax.numpy as jnp
from jax import lax
from jax.experimental import pallas as pl
from jax.experimental.pallas import tpu as pltpu
from jax.experimental.pallas import tpu_sc as plsc

CHUNK = 128
HALF = CHUNK // 2


@functools.lru_cache(maxsize=None)
def _build_gather(B, DT, V2):
    info = plsc.get_sparse_core_info()
    NC, NS = info.num_cores, info.num_subcores
    NW = NC * NS
    per_w = B // NW
    assert per_w * NW == B and per_w % CHUNK == 0
    k = per_w // CHUNK
    assert k % 2 == 0
    m = k // 2 - 1

    mesh = plsc.VectorSubcoreMesh(core_axis_name="c", subcore_axis_name="s")

    @functools.partial(
        pl.kernel,
        mesh=mesh,
        out_type=jax.ShapeDtypeStruct((B // 2, 2 * DT), jnp.float32),
        scratch_types=[
            pltpu.VMEM((k, 2, HALF), jnp.int32),
            pltpu.VMEM((2, HALF, DT), jnp.float32),
            pltpu.VMEM((2, HALF, DT), jnp.float32),
            pltpu.SemaphoreType.DMA,
            pltpu.SemaphoreType.DMA,
            pltpu.SemaphoreType.DMA,
            pltpu.SemaphoreType.DMA,
        ],
        compiler_params=pltpu.CompilerParams(use_tc_tiling_on_sc=False),
    )
    def sc_kernel(idx_hbm, table_hbm, g_hbm, idx_v, buf0, buf1, si0, si1, so0, so1):
        wid = lax.axis_index("s") * NC + lax.axis_index("c")
        base = wid * per_w // 2
        pltpu.sync_copy(idx_hbm.at[wid], idx_v)

        def start_in(j, buf, sem):
            pltpu.async_copy(table_hbm.at[idx_v.at[j, 0]], buf.at[0], sem)
            pltpu.async_copy(table_hbm.at[idx_v.at[j, 1]], buf.at[1], sem)

        def start_out(j, buf, sem):
            rows = pl.ds(base + j * HALF, HALF)
            pltpu.async_copy(buf.at[0], g_hbm.at[rows, pl.ds(0, DT)], sem)
            pltpu.async_copy(buf.at[1], g_hbm.at[rows, pl.ds(DT, DT)], sem)

        def wait_in(sem, buf):
            for h in (0, 1):
                pltpu.make_async_copy(
                    table_hbm.at[idx_v.at[0, 0]], buf.at[h], sem
                ).wait()

        def wait_out(sem, buf):
            for h in (0, 1):
                pltpu.make_async_copy(
                    buf.at[h], g_hbm.at[pl.ds(base, HALF), pl.ds(0, DT)], sem
                ).wait()

        start_in(0, buf0, si0)
        start_in(1, buf1, si1)

        def body(i, _):
            j = 2 * i
            wait_in(si0, buf0)
            start_out(j, buf0, so0)
            wait_in(si1, buf1)
            start_out(j + 1, buf1, so1)
            wait_out(so0, buf0)
            start_in(j + 2, buf0, si0)
            wait_out(so1, buf1)
            start_in(j + 3, buf1, si1)
            return ()

        lax.fori_loop(0, m, body, (), unroll=False)

        wait_in(si0, buf0)
        start_out(k - 2, buf0, so0)
        wait_in(si1, buf1)
        start_out(k - 1, buf1, so1)
        wait_out(so0, buf0)
        wait_out(so1, buf1)

    return sc_kernel, NW, k


@functools.lru_cache(maxsize=None)
def _build_concat(S0, S1, DE, DT, BS, S1B):
    DO = DE + DT

    def concat_body(elmo_ref, g_ref, out_ref):
        e = elmo_ref[...]
        et = jnp.transpose(e, (0, 2, 1))
        g = g_ref[...]
        gt = jnp.transpose(g, (0, 2, 1))
        gt = gt.reshape(S1B, DT, BS)
        out_ref[...] = jnp.concatenate((et, gt), axis=1)

    return pl.pallas_call(
        concat_body,
        grid=(S0 // BS, S1 // S1B),
        in_specs=[
            pl.BlockSpec((S1B, BS, DE), lambda i, j: (j, i, 0)),
            pl.BlockSpec((S1B // 2, BS, 2 * DT), lambda i, j: (j, i, 0)),
        ],
        out_specs=pl.BlockSpec((S1B, DO, BS), lambda i, j: (j, 0, i)),
        out_shape=jax.ShapeDtypeStruct((S1, DO, S0), jnp.float32),
    )


@functools.lru_cache(maxsize=None)
def _build_detile(V, DT, BC):
    import math

    R = BC // 2
    grid = math.ceil(V / BC)

    def detile_body(t_ref, out_ref):
        a = t_ref[...]
        at = jnp.transpose(a, (1, 0))
        out_ref[...] = jnp.concatenate((at[:R], at[R:]), axis=1)

    return pl.pallas_call(
        detile_body,
        grid=(grid,),
        in_specs=[pl.BlockSpec((DT, BC), lambda i: (0, i))],
        out_specs=pl.BlockSpec((R, 2 * DT), lambda i: (i, 0)),
        out_shape=jax.ShapeDtypeStruct((grid * R, 2 * DT), jnp.float32),
    )


def kernel(elmo_emb, inp, table):
    S0, S1, DE = elmo_emb.shape
    V, DT = table.shape
    B = S0 * S1
    sc_gather, NW, k = _build_gather(B, DT, V)
    BC = 2048
    detile = _build_detile(V, DT, BC)
    tl = detile(table.transpose(1, 0))
    table_lin = tl.reshape(tl.shape[0] * 2, DT)
    ii = inp.astype(jnp.int32)
    loc = ii & (BC - 1)
    perm = (ii & ~(BC - 1)) + jnp.where(
        loc < BC // 2, loc << 1, ((loc - BC // 2) << 1) + 1
    )
    ip = perm.transpose(1, 0).reshape(S1 // 2, 2, S0)
    ev = ip[:, 0, :].reshape(NW, k, HALF)
    od = ip[:, 1, :].reshape(NW, k, HALF)
    idx = jnp.stack((ev, od), axis=2)
    g = sc_gather(idx, table_lin)
    g3 = g.reshape(S1 // 2, S0, 2 * DT)
    concat = _build_concat(S0, S1, DE, DT, 128, 10)
    elmo_t = elmo_emb.transpose(1, 0, 2)
    out_t = concat(elmo_t, g3)
    return out_t.transpose(2, 0, 1)
